# baseline (speedup 1.0000x reference)
import functools

import jax
import jax.numpy as jnp
from jax import lax
from jax.experimental import pallas as pl
from jax.experimental.pallas import tpu as pltpu
from jax.experimental.pallas import tpu_sc as plsc


def _ring(table_hbm, out_hbm, buf, in_sem, out_sem, base, chunk, nbuf, look,
          nchunks):
    def load(j):
        pltpu.async_copy(
            table_hbm.at[pl.ds(base + j * chunk, chunk)],
            buf.at[j % nbuf],
            in_sem,
        )

    def store(j):
        pltpu.async_copy(
            buf.at[j % nbuf],
            out_hbm.at[0, pl.ds(base + j * chunk, chunk)],
            out_sem,
        )

    def drain_in(j):
        pltpu.make_async_copy(
            table_hbm.at[pl.ds(base, chunk)], buf.at[j % nbuf], in_sem
        ).wait()

    def drain_out(j):
        pltpu.make_async_copy(
            buf.at[j % nbuf], out_hbm.at[0, pl.ds(base, chunk)], out_sem
        ).wait()

    for j in range(min(look, nchunks)):
        load(j)
    for i in range(nchunks):
        d = i - (nbuf - look)
        if d >= 0:
            drain_out(d)
        j = i + look
        if j < nchunks:
            load(j)
        drain_in(i)
        store(i)
    for d in range(max(0, nchunks - (nbuf - look)), nchunks):
        drain_out(d)


def _make_copy_kernel(L, D, dtype):
    NC, NS = 2, 16
    rows_per_core = L // NC
    s_rows = 1088
    s_chunk, s_nbuf, s_look = 64, 8, 4
    s_nchunks = s_rows // s_chunk
    t_rows = (rows_per_core - s_rows) // (NS - 1)
    t_chunk, t_nbuf, t_look = 16, 6, 3
    t_nchunks = t_rows // t_chunk

    mesh = plsc.VectorSubcoreMesh(
        core_axis_name="c", subcore_axis_name="s", num_cores=NC
    )

    @functools.partial(
        pl.kernel,
        mesh=mesh,
        out_type=jax.ShapeDtypeStruct((1, L, D), dtype),
        scratch_types=[
            pltpu.VMEM((t_nbuf, t_chunk, D), dtype),
            pltpu.VMEM_SHARED((s_nbuf, s_chunk, D), dtype),
            pltpu.SemaphoreType.DMA,
            pltpu.SemaphoreType.DMA,
        ],
    )
    def copy_k(table_hbm, out_hbm, tbuf, sbuf, in_sem, out_sem):
        cid = lax.axis_index("c")
        sid = lax.axis_index("s")
        core_base = cid * rows_per_core

        @pl.when(sid == NS - 1)
        def _spmem_path():
            _ring(table_hbm, out_hbm, sbuf, in_sem, out_sem, core_base,
                  s_chunk, s_nbuf, s_look, s_nchunks)

        @pl.when(sid < NS - 1)
        def _tile_path():
            base = core_base + s_rows + sid * t_rows
            _ring(table_hbm, out_hbm, tbuf, in_sem, out_sem, base,
                  t_chunk, t_nbuf, t_look, t_nchunks)

    return copy_k


def kernel(hidden_states, position_table):
    L = hidden_states.shape[1]
    D = position_table.shape[1]
    copy_k = _make_copy_kernel(L, D, position_table.dtype)
    return copy_k(position_table)


# speedup vs baseline: 1.0028x; 1.0028x over previous
import functools

import jax
import jax.numpy as jnp
from jax import lax
from jax.experimental import pallas as pl
from jax.experimental.pallas import tpu as pltpu
from jax.experimental.pallas import tpu_sc as plsc


def _make_copy_kernel(L, D, dtype, num_cores):
    rows_per_c = L // num_cores
    chunk = 256
    nbuf = 4
    look = 2
    nchunks = rows_per_c // chunk

    mesh = plsc.ScalarSubcoreMesh(axis_name="c", num_cores=num_cores)

    @functools.partial(
        pl.kernel,
        mesh=mesh,
        out_type=jax.ShapeDtypeStruct((1, L, D), dtype),
        scratch_types=[
            pltpu.VMEM_SHARED((nbuf, chunk, D), dtype),
            pltpu.SemaphoreType.DMA,
            pltpu.SemaphoreType.DMA,
        ],
    )
    def copy_k(table_hbm, out_hbm, buf, in_sem, out_sem):
        base = lax.axis_index("c") * rows_per_c

        def load(j):
            pltpu.async_copy(
                table_hbm.at[pl.ds(base + j * chunk, chunk)],
                buf.at[j % nbuf],
                in_sem,
            )

        def store(j):
            pltpu.async_copy(
                buf.at[j % nbuf],
                out_hbm.at[0, pl.ds(base + j * chunk, chunk)],
                out_sem,
            )

        def drain_in(j):
            pltpu.make_async_copy(
                table_hbm.at[pl.ds(base, chunk)], buf.at[j % nbuf], in_sem
            ).wait()

        def drain_out(j):
            pltpu.make_async_copy(
                buf.at[j % nbuf], out_hbm.at[0, pl.ds(base, chunk)], out_sem
            ).wait()

        for j in range(min(look, nchunks)):
            load(j)
        for i in range(nchunks):
            d = i - (nbuf - look)
            if d >= 0:
                drain_out(d)
            j = i + look
            if j < nchunks:
                load(j)
            drain_in(i)
            store(i)
        for d in range(max(0, nchunks - (nbuf - look)), nchunks):
            drain_out(d)

    return copy_k


def kernel(hidden_states, position_table):
    L = hidden_states.shape[1]
    D = position_table.shape[1]
    copy_k = _make_copy_kernel(L, D, position_table.dtype, 2)
    return copy_k(position_table)


# speedup vs baseline: 1.0199x; 1.0170x over previous
import functools

import jax
import jax.numpy as jnp
from jax import lax
from jax.experimental import pallas as pl
from jax.experimental.pallas import tpu as pltpu
from jax.experimental.pallas import tpu_sc as plsc


def _make_copy_kernel(L, D, dtype, num_cores):
    rows_per_c = L // num_cores
    chunk = 128
    nbuf = 8
    look = 6
    nchunks = rows_per_c // chunk

    mesh = plsc.ScalarSubcoreMesh(axis_name="c", num_cores=num_cores)

    @functools.partial(
        pl.kernel,
        mesh=mesh,
        out_type=jax.ShapeDtypeStruct((1, L, D), dtype),
        scratch_types=[
            pltpu.VMEM_SHARED((nbuf, chunk, D), dtype),
            pltpu.SemaphoreType.DMA,
            pltpu.SemaphoreType.DMA,
        ],
    )
    def copy_k(table_hbm, out_hbm, buf, in_sem, out_sem):
        base = lax.axis_index("c") * rows_per_c

        def load(j):
            pltpu.async_copy(
                table_hbm.at[pl.ds(base + j * chunk, chunk)],
                buf.at[j % nbuf],
                in_sem,
            )

        def store(j):
            pltpu.async_copy(
                buf.at[j % nbuf],
                out_hbm.at[0, pl.ds(base + j * chunk, chunk)],
                out_sem,
            )

        def drain_in(j):
            pltpu.make_async_copy(
                table_hbm.at[pl.ds(base, chunk)], buf.at[j % nbuf], in_sem
            ).wait()

        def drain_out(j):
            pltpu.make_async_copy(
                buf.at[j % nbuf], out_hbm.at[0, pl.ds(base, chunk)], out_sem
            ).wait()

        for j in range(min(look, nchunks)):
            load(j)
        for i in range(nchunks):
            d = i - (nbuf - look)
            if d >= 0:
                drain_out(d)
            j = i + look
            if j < nchunks:
                load(j)
            drain_in(i)
            store(i)
        for d in range(max(0, nchunks - (nbuf - look)), nchunks):
            drain_out(d)

    return copy_k


def kernel(hidden_states, position_table):
    L = hidden_states.shape[1]
    D = position_table.shape[1]
    copy_k = _make_copy_kernel(L, D, position_table.dtype, 2)
    return copy_k(position_table)


# speedup vs baseline: 1.0334x; 1.0133x over previous
import functools

import jax
import jax.numpy as jnp
from jax import lax
from jax.experimental import pallas as pl
from jax.experimental.pallas import tpu as pltpu
from jax.experimental.pallas import tpu_sc as plsc


def _make_copy_kernel(L, D, dtype, num_cores):
    rows_per_c = L // num_cores
    chunk = 128
    nbuf = 8
    look = 4
    nchunks = rows_per_c // chunk

    mesh = plsc.ScalarSubcoreMesh(axis_name="c", num_cores=num_cores)

    @functools.partial(
        pl.kernel,
        mesh=mesh,
        out_type=jax.ShapeDtypeStruct((1, L, D), dtype),
        scratch_types=[
            pltpu.VMEM_SHARED((nbuf, chunk, D), dtype),
            pltpu.SemaphoreType.DMA,
            pltpu.SemaphoreType.DMA,
        ],
    )
    def copy_k(table_hbm, out_hbm, buf, in_sem, out_sem):
        base = lax.axis_index("c") * rows_per_c

        def load(j):
            pltpu.async_copy(
                table_hbm.at[pl.ds(base + j * chunk, chunk)],
                buf.at[j % nbuf],
                in_sem,
            )

        def store(j):
            pltpu.async_copy(
                buf.at[j % nbuf],
                out_hbm.at[0, pl.ds(base + j * chunk, chunk)],
                out_sem,
            )

        def drain_in(j):
            pltpu.make_async_copy(
                table_hbm.at[pl.ds(base, chunk)], buf.at[j % nbuf], in_sem
            ).wait()

        def drain_out(j):
            pltpu.make_async_copy(
                buf.at[j % nbuf], out_hbm.at[0, pl.ds(base, chunk)], out_sem
            ).wait()

        for j in range(min(look, nchunks)):
            load(j)
        for i in range(nchunks):
            d = i - (nbuf - look)
            if d >= 0:
                drain_out(d)
            j = i + look
            if j < nchunks:
                load(j)
            drain_in(i)
            store(i)
        for d in range(max(0, nchunks - (nbuf - look)), nchunks):
            drain_out(d)

    return copy_k


def kernel(hidden_states, position_table):
    L = hidden_states.shape[1]
    D = position_table.shape[1]
    copy_k = _make_copy_kernel(L, D, position_table.dtype, 2)
    return copy_k(position_table)


# speedup vs baseline: 1.0453x; 1.0115x over previous
import functools

import jax
import jax.numpy as jnp
from jax import lax
from jax.experimental import pallas as pl
from jax.experimental.pallas import tpu as pltpu
from jax.experimental.pallas import tpu_sc as plsc


def _make_copy_kernel(L, D, dtype, num_cores):
    rows_per_c = L // num_cores
    chunk = 64
    nbuf = 16
    look = 8
    nchunks = rows_per_c // chunk

    mesh = plsc.ScalarSubcoreMesh(axis_name="c", num_cores=num_cores)

    @functools.partial(
        pl.kernel,
        mesh=mesh,
        out_type=jax.ShapeDtypeStruct((1, L, D), dtype),
        scratch_types=[
            pltpu.VMEM_SHARED((nbuf, chunk, D), dtype),
            pltpu.SemaphoreType.DMA,
            pltpu.SemaphoreType.DMA,
        ],
    )
    def copy_k(table_hbm, out_hbm, buf, in_sem, out_sem):
        base = lax.axis_index("c") * rows_per_c

        def load(j):
            pltpu.async_copy(
                table_hbm.at[pl.ds(base + j * chunk, chunk)],
                buf.at[j % nbuf],
                in_sem,
            )

        def store(j):
            pltpu.async_copy(
                buf.at[j % nbuf],
                out_hbm.at[0, pl.ds(base + j * chunk, chunk)],
                out_sem,
            )

        def drain_in(j):
            pltpu.make_async_copy(
                table_hbm.at[pl.ds(base, chunk)], buf.at[j % nbuf], in_sem
            ).wait()

        def drain_out(j):
            pltpu.make_async_copy(
                buf.at[j % nbuf], out_hbm.at[0, pl.ds(base, chunk)], out_sem
            ).wait()

        for j in range(min(look, nchunks)):
            load(j)
        for i in range(nchunks):
            d = i - (nbuf - look)
            if d >= 0:
                drain_out(d)
            j = i + look
            if j < nchunks:
                load(j)
            drain_in(i)
            store(i)
        for d in range(max(0, nchunks - (nbuf - look)), nchunks):
            drain_out(d)

    return copy_k


def kernel(hidden_states, position_table):
    L = hidden_states.shape[1]
    D = position_table.shape[1]
    copy_k = _make_copy_kernel(L, D, position_table.dtype, 2)
    return copy_k(position_table)
